# baseline (speedup 1.0000x reference)
import jax
import jax.numpy as jnp
from jax.experimental import pallas as pl
from jax.experimental.pallas import tpu as pltpu

_N_CHUNKS = 8


def _copy_body(x_ref, o_ref, sem):
    rows = x_ref.shape[0]
    rpc = rows // _N_CHUNKS
    for i in range(_N_CHUNKS):
        pltpu.make_async_copy(
            x_ref.at[pl.ds(i * rpc, rpc)],
            o_ref.at[pl.ds(i * rpc, rpc)],
            sem,
        ).start()
    for i in range(_N_CHUNKS):
        pltpu.make_async_copy(
            x_ref.at[pl.ds(i * rpc, rpc)],
            o_ref.at[pl.ds(i * rpc, rpc)],
            sem,
        ).wait()


def kernel(inputs_embeds, embed_table):
    del embed_table
    b, s, h = inputs_embeds.shape
    x = inputs_embeds.reshape(b * s, h)
    rows = b * s
    out = pl.pallas_call(
        _copy_body,
        in_specs=[pl.BlockSpec(memory_space=pl.ANY)],
        out_specs=pl.BlockSpec(memory_space=pl.ANY),
        out_shape=jax.ShapeDtypeStruct((rows, h), x.dtype),
        scratch_shapes=[pltpu.SemaphoreType.DMA],
    )(x)
    return out.reshape(b, s, h)


# speedup vs baseline: 38.9677x; 38.9677x over previous
import functools
import jax
import jax.numpy as jnp
from jax import lax
from jax.experimental import pallas as pl
from jax.experimental.pallas import tpu as pltpu
from jax.experimental.pallas import tpu_sc as plsc

ROWS = 32768
H = 2048
NC, NS = 2, 16
NW = NC * NS
RPW = ROWS // NW
CH = 16
NCHUNKS = RPW // CH

_mesh = plsc.VectorSubcoreMesh(core_axis_name="c", subcore_axis_name="s")


@functools.partial(
    pl.kernel,
    out_type=jax.ShapeDtypeStruct((ROWS, H), jnp.float32),
    mesh=_mesh,
    scratch_types=[
        pltpu.VMEM((2, CH, H), jnp.float32),
        pltpu.SemaphoreType.DMA,
        pltpu.SemaphoreType.DMA,
    ],
)
def _sc_copy(x_hbm, o_hbm, buf, ld_sem, st_sem):
    wid = lax.axis_index("s") * NC + lax.axis_index("c")
    base = wid * RPW

    def ld(g, slot):
        return pltpu.make_async_copy(
            x_hbm.at[pl.ds(base + g * CH, CH)], buf.at[slot], ld_sem)

    def st(g, slot):
        return pltpu.make_async_copy(
            buf.at[slot], o_hbm.at[pl.ds(base + g * CH, CH)], st_sem)

    ld(0, 0).start()
    for g in range(NCHUNKS):
        slot = g % 2
        ld(g, slot).wait()
        st(g, slot).start()
        if g + 1 < NCHUNKS:
            if g >= 1:
                st(g - 1, (g - 1) % 2).wait()
            ld(g + 1, (g + 1) % 2).start()
    st(NCHUNKS - 1, (NCHUNKS - 1) % 2).wait()


def kernel(inputs_embeds, embed_table):
    del embed_table
    b, s, h = inputs_embeds.shape
    x = inputs_embeds.reshape(b * s, h)
    out = _sc_copy(x)
    return out.reshape(b, s, h)


# speedup vs baseline: 48.4621x; 1.2437x over previous
import jax
import jax.numpy as jnp
from jax.experimental import pallas as pl
from jax.experimental.pallas import tpu as pltpu

_CH = 512
_NBUF = 8


def _copy_body(x_ref, o_ref, buf, ld_sems, st_sems):
    rows = x_ref.shape[0]
    n = rows // _CH

    def ld(g):
        slot = g % _NBUF
        return pltpu.make_async_copy(
            x_ref.at[pl.ds(g * _CH, _CH)], buf.at[slot], ld_sems.at[slot])

    def st(g):
        slot = g % _NBUF
        return pltpu.make_async_copy(
            buf.at[slot], o_ref.at[pl.ds(g * _CH, _CH)], st_sems.at[slot])

    for g in range(min(_NBUF, n)):
        ld(g).start()
    for g in range(n):
        ld(g).wait()
        st(g).start()
        nxt = g + _NBUF
        if nxt < n:
            st(g).wait()
            ld(nxt).start()
    for g in range(max(0, n - _NBUF), n):
        st(g).wait()


def kernel(inputs_embeds, embed_table):
    del embed_table
    b, s, h = inputs_embeds.shape
    x = inputs_embeds.reshape(b * s, h)
    rows = b * s
    out = pl.pallas_call(
        _copy_body,
        in_specs=[pl.BlockSpec(memory_space=pl.ANY)],
        out_specs=pl.BlockSpec(memory_space=pl.ANY),
        out_shape=jax.ShapeDtypeStruct((rows, h), x.dtype),
        scratch_shapes=[
            pltpu.VMEM((_NBUF, _CH, h), jnp.float32),
            pltpu.SemaphoreType.DMA((_NBUF,)),
            pltpu.SemaphoreType.DMA((_NBUF,)),
        ],
    )(x)
    return out.reshape(b, s, h)


# speedup vs baseline: 49.1813x; 1.0148x over previous
import jax
import jax.numpy as jnp
from jax.experimental import pallas as pl
from jax.experimental.pallas import tpu as pltpu

_CH = 512
_NBUF = 8


def _copy_body(x_ref, o_ref, buf, ld_sems, st_sems):
    rows = x_ref.shape[0]
    n = rows // _CH

    def ld(g):
        slot = g % _NBUF
        return pltpu.make_async_copy(
            x_ref.at[pl.ds(g * _CH, _CH)], buf.at[slot], ld_sems.at[slot])

    def st(g):
        slot = g % _NBUF
        return pltpu.make_async_copy(
            buf.at[slot], o_ref.at[pl.ds(g * _CH, _CH)], st_sems.at[slot])

    k = _NBUF // 2
    for g in range(min(k, n)):
        ld(g).start()
    for g in range(n):
        ld(g).wait()
        st(g).start()
        nxt = g + k
        if nxt < n:
            prev = nxt - _NBUF
            if prev >= 0:
                st(prev).wait()
            ld(nxt).start()
    for g in range(max(0, n - _NBUF), n):
        st(g).wait()


def kernel(inputs_embeds, embed_table):
    del embed_table
    b, s, h = inputs_embeds.shape
    x = inputs_embeds.reshape(b * s, h)
    rows = b * s
    out = pl.pallas_call(
        _copy_body,
        in_specs=[pl.BlockSpec(memory_space=pl.ANY)],
        out_specs=pl.BlockSpec(memory_space=pl.ANY),
        out_shape=jax.ShapeDtypeStruct((rows, h), x.dtype),
        scratch_shapes=[
            pltpu.VMEM((_NBUF, _CH, h), jnp.float32),
            pltpu.SemaphoreType.DMA((_NBUF,)),
            pltpu.SemaphoreType.DMA((_NBUF,)),
        ],
    )(x)
    return out.reshape(b, s, h)
